# baseline (speedup 1.0000x reference)
import functools

import jax
import jax.numpy as jnp
from jax import lax
from jax.experimental import pallas as pl
from jax.experimental.pallas import tpu as pltpu
from jax.experimental.pallas import tpu_sc as plsc

N = 10000
NP = 10240
E = 320000
EP = 323584
B = 64
D_IN = 128
LAT = 128
HID = 256
MLP_HID = 512
OUT_DIM = 1275

NC, NS = 2, 16
NT = NP // NS
NH = NP // 2

_R = 256
_GI = NP // _R


def _sc_mesh():
    return plsc.VectorSubcoreMesh(
        core_axis_name="c", subcore_axis_name="s", num_cores=NC, num_subcores=NS
    )


_DEG_ET = EP // NS


@functools.partial(
    pl.kernel,
    out_type=jax.ShapeDtypeStruct((NC * NS, NH), jnp.float32),
    mesh=_sc_mesh(),
    compiler_params=pltpu.CompilerParams(needs_layout_passes=False),
    scratch_types=[
        pltpu.VMEM((_DEG_ET,), jnp.int32),
        pltpu.VMEM((NS * NH,), jnp.float32),
        pltpu.VMEM((NH,), jnp.float32),
    ],
)
def _deg_kernel(dst_hbm, out_hbm, dstbuf, cnt, red):
    c = lax.axis_index("c")
    t = lax.axis_index("s")
    lo = c * NH
    zeros16 = jnp.zeros((16,), jnp.float32)
    ones16 = jnp.ones((16,), jnp.float32)
    lane = lax.iota(jnp.int32, 16)

    def zbody(i, carry):
        cnt[pl.ds(i * 16, 16)] = zeros16
        return carry

    lax.fori_loop(0, NS * NH // 16, zbody, 0)

    pltpu.sync_copy(dst_hbm.at[pl.ds(t * _DEG_ET, _DEG_ET)], dstbuf)

    lane_off = lane * NH

    def body(j, carry):
        idx = dstbuf[pl.ds(j * 16, 16)]
        m = (idx >= lo) & (idx < lo + NH)
        loc = jnp.where(m, idx - lo, 0) + lane_off
        plsc.addupdate_scatter(cnt, [loc], ones16, mask=m)
        return carry

    lax.fori_loop(0, _DEG_ET // 16, body, 0)

    def rbody(i, carry):
        acc = cnt[pl.ds(i * 16, 16)]
        for r in range(1, NS):
            acc = acc + cnt[pl.ds(r * NH + i * 16, 16)]
        red[pl.ds(i * 16, 16)] = acc
        return carry

    lax.fori_loop(0, NH // 16, rbody, 0)

    pltpu.sync_copy(red, out_hbm.at[c * NS + t])


_K = 64
_CPS = 4
_NBQ = 4
_NBR = 5


def _make_conv(epc, f, init_split):
    et = epc // NS
    ch = et // _K
    nq = ch // _CPS
    assert ch % _CPS == 0 and nq >= 5

    @functools.partial(
        pl.kernel,
        out_type=jax.ShapeDtypeStruct((NC * NP, f), jnp.float32),
        mesh=_sc_mesh(),
        compiler_params=pltpu.CompilerParams(needs_layout_passes=False),
        scratch_types=[
            pltpu.VMEM((_NBQ, _CPS, 2, 1, _K), jnp.int32),
            pltpu.VMEM((_NBR, _K, f), jnp.float32),
            pltpu.VMEM_SHARED((NP, f), jnp.float32),
            pltpu.SemaphoreType.DMA((_NBQ,)),
            pltpu.SemaphoreType.DMA((_NBR,)),
            pltpu.SemaphoreType.DMA((_NBR,)),
        ],
    )
    def conv(g_hbm, ginit_hbm, idx_hbm, out_hbm,
             iring, rows, acc, isem, gsem, ssem):
        c = lax.axis_index("c")
        t = lax.axis_index("s")
        ioff = (c * NP if init_split else 0) + t * NT
        pltpu.sync_copy(ginit_hbm.at[pl.ds(ioff, NT)], acc.at[pl.ds(t * NT, NT)])
        sbase = (c * epc + t * et) // (_K * _CPS)

        def issue_i(q):
            b = lax.rem(q, _NBQ)
            pltpu.async_copy(idx_hbm.at[sbase + q], iring.at[b], isem.at[b])

        def wait_i(q):
            b = lax.rem(q, _NBQ)
            pltpu.make_async_copy(idx_hbm.at[sbase + q], iring.at[b],
                                  isem.at[b]).wait()

        def _sidx(k):
            return iring.at[lax.rem(k // _CPS, _NBQ), lax.rem(k, _CPS), 0, 0]

        def _didx(k):
            return iring.at[lax.rem(k // _CPS, _NBQ), lax.rem(k, _CPS), 1, 0]

        def issue_g(k):
            br = lax.rem(k, _NBR)
            pltpu.async_copy(g_hbm.at[_sidx(k)], rows.at[br], gsem.at[br])

        def wait_g(k):
            br = lax.rem(k, _NBR)
            pltpu.make_async_copy(g_hbm.at[_sidx(k)], rows.at[br],
                                  gsem.at[br]).wait()

        def issue_s(k):
            br = lax.rem(k, _NBR)
            pltpu.async_copy(rows.at[br], acc.at[_didx(k)], ssem.at[br],
                             add=True)

        def wait_s(k):
            br = lax.rem(k, _NBR)
            pltpu.make_async_copy(rows.at[br], acc.at[_didx(k)],
                                  ssem.at[br]).wait()

        def chunk(q, j, do_wait_s, do_issue_g, do_issue_i):
            k = q * _CPS + j
            wait_g(k)
            issue_s(k)
            if do_wait_s:
                wait_s(k - 2)
            if do_issue_g:
                if j == 1:
                    wait_i(q + 1)
                issue_g(k + 3)
            if do_issue_i and j == 2:
                issue_i(q + 3)

        issue_i(0)
        issue_i(1)
        issue_i(2)
        wait_i(0)
        issue_g(0)
        issue_g(1)
        issue_g(2)
        plsc.subcore_barrier()

        chunk(0, 0, False, True, False)
        chunk(0, 1, False, True, False)
        chunk(0, 2, True, True, True)
        chunk(0, 3, True, True, False)

        def body(q, carry):
            for j in range(_CPS):
                chunk(q, j, True, True, True)
            return carry

        lax.fori_loop(1, nq - 3, body, 0)

        qe = nq - 3
        for j in range(_CPS):
            chunk(qe, j, True, True, False)
        qe = nq - 2
        for j in range(_CPS):
            chunk(qe, j, True, True, False)
        qe = nq - 1
        chunk(qe, 0, True, True, False)
        chunk(qe, 1, True, False, False)
        chunk(qe, 2, True, False, False)
        chunk(qe, 3, True, False, False)
        wait_s(ch - 2)
        wait_s(ch - 1)
        plsc.subcore_barrier()
        pltpu.sync_copy(acc.at[pl.ds(t * NT, NT)],
                        out_hbm.at[pl.ds(c * NP + t * NT, NT)])

    return conv


EP2 = 327680
_conv1 = _make_conv(EP, D_IN, True)
_conv2 = _make_conv(EP2 // 2, LAT, False)


def _tc1_body(xb, w1b, dpb, g1b, degb):
    c = pl.program_id(1)
    deg = jnp.sum(dpb[...], axis=0) + 1.0
    dinv = lax.rsqrt(deg)
    h = jnp.dot(xb[...], w1b[...], preferred_element_type=jnp.float32)
    g1b[...] = h * dinv[:, None]

    @pl.when(c == 0)
    def _():
        degb[...] = deg


def _tc1(x_p, w1, deg_parts):
    return pl.pallas_call(
        _tc1_body,
        grid=(_GI, NC),
        in_specs=[
            pl.BlockSpec((_R, D_IN), lambda i, c: (i, 0)),
            pl.BlockSpec((D_IN, LAT), lambda i, c: (0, c)),
            pl.BlockSpec((NS, _R), lambda i, c: (i // 20, i % 20)),
        ],
        out_specs=[
            pl.BlockSpec((_R, LAT), lambda i, c: (c * _GI + i, 0)),
            pl.BlockSpec((_R,), lambda i, c: (i,)),
        ],
        out_shape=[
            jax.ShapeDtypeStruct((NC * NP, LAT), jnp.float32),
            jax.ShapeDtypeStruct((NP,), jnp.float32),
        ],
    )(x_p, w1, deg_parts)


def _tc2_body(aab, abb, degb, b1b, w2b, g2b, g2hb):
    dinv = lax.rsqrt(degb[...])
    a = jnp.concatenate([aab[...], abb[...]], axis=1)
    out1 = jnp.maximum(a * dinv[:, None] + b1b[...][None, :], 0.0)
    h2 = jnp.dot(out1, w2b[...], preferred_element_type=jnp.float32)
    g2 = h2 * dinv[:, None]
    g2b[...] = g2
    g2hb[...] = 0.5 * g2


def _tc2(acc1, deg, b1, w2):
    return pl.pallas_call(
        _tc2_body,
        grid=(_GI,),
        in_specs=[
            pl.BlockSpec((_R, D_IN), lambda i: (i, 0)),
            pl.BlockSpec((_R, D_IN), lambda i: (_GI + i, 0)),
            pl.BlockSpec((_R,), lambda i: (i,)),
            pl.BlockSpec((HID,), lambda i: (0,)),
            pl.BlockSpec((HID, LAT), lambda i: (0, 0)),
        ],
        out_specs=[
            pl.BlockSpec((_R, LAT), lambda i: (i, 0)),
            pl.BlockSpec((_R, LAT), lambda i: (i, 0)),
        ],
        out_shape=[
            jax.ShapeDtypeStruct((NP, LAT), jnp.float32),
            jax.ShapeDtypeStruct((NP, LAT), jnp.float32),
        ],
    )(acc1, acc1, deg, b1, w2)


def _tcf_body(acc2r, degr, b2r, startsr, wd1r, bd1r, wd2r, bd2r, outr, hfin):
    dinv = lax.rsqrt(degr[...])
    a = acc2r[pl.ds(0, NP), :] + acc2r[pl.ds(NP, NP), :]
    hfin[...] = a * dinv[:, None] + b2r[...][None, :]

    neg = jnp.float32(-3.0e38)

    def seg(b, z):
        s0 = startsr[b]
        s1 = startsr[b + 1]
        j0 = s0 // 8
        nb = (s1 - j0 * 8 + 7) // 8

        def blk(j, acc8):
            off = pl.multiple_of((j0 + j) * 8, 8)
            rows = hfin[pl.ds(off, 8), :]
            rid = off + lax.broadcasted_iota(jnp.int32, (8, 1), 0)
            keep = (rid >= s0) & (rid < s1)
            return jnp.maximum(acc8, jnp.where(keep, rows, neg))

        acc8 = lax.fori_loop(0, nb, blk, jnp.full((8, LAT), neg, jnp.float32))
        zrow = jnp.max(acc8, axis=0)
        sel = lax.broadcasted_iota(jnp.int32, (B, 1), 0) == b
        return jnp.where(sel, zrow[None, :], z)

    z = lax.fori_loop(0, B, seg, jnp.full((B, LAT), neg, jnp.float32))
    y = jnp.maximum(
        jnp.dot(z, wd1r[...], preferred_element_type=jnp.float32)
        + bd1r[...][None, :], 0.0)
    o = (jnp.dot(y, wd2r[...], preferred_element_type=jnp.float32)
         + bd2r[...][None, :])
    outr[...] = jax.nn.sigmoid(o)


def _tcf(acc2, deg, b2, starts, wd1, bd1, wd2, bd2):
    return pl.pallas_call(
        _tcf_body,
        in_specs=[
            pl.BlockSpec(memory_space=pltpu.VMEM),
            pl.BlockSpec(memory_space=pltpu.VMEM),
            pl.BlockSpec(memory_space=pltpu.VMEM),
            pl.BlockSpec(memory_space=pltpu.SMEM),
            pl.BlockSpec(memory_space=pltpu.VMEM),
            pl.BlockSpec(memory_space=pltpu.VMEM),
            pl.BlockSpec(memory_space=pltpu.VMEM),
            pl.BlockSpec(memory_space=pltpu.VMEM),
        ],
        out_shape=jax.ShapeDtypeStruct((B, OUT_DIM), jnp.float32),
        scratch_shapes=[pltpu.VMEM((NP, LAT), jnp.float32)],
    )(acc2, deg, b2, starts, wd1, bd1, wd2, bd2)


def kernel(x, edge_index, batch, W1, b1, W2, b2, Wd1, bd1, Wd2, bd2):
    src = edge_index[0]
    dst = edge_index[1]
    pad = EP - E
    src_p = jnp.concatenate([src, jnp.zeros((pad,), jnp.int32)])
    trash = N + jnp.arange(pad, dtype=jnp.int32) % (NP - N)
    dst_p = jnp.concatenate([dst, trash])
    src2 = jnp.concatenate([src_p, src_p + NP])
    dst2 = jnp.concatenate([dst_p, dst_p])
    x_p = jnp.pad(x, ((0, NP - N), (0, 0)))
    starts = jnp.searchsorted(batch, jnp.arange(B + 1, dtype=jnp.int32)
                              ).astype(jnp.int32)

    pad2 = EP2 - E
    src_q = jnp.concatenate([src, jnp.zeros((pad2,), jnp.int32)])
    dst_q = jnp.concatenate(
        [dst, N + jnp.arange(pad2, dtype=jnp.int32) % (NP - N)])

    idx1 = jnp.concatenate([src2.reshape(-1, 1, 1, 64),
                            dst2.reshape(-1, 1, 1, 64)],
                           axis=1).reshape(-1, 4, 2, 1, 64)
    idx2 = jnp.concatenate([src_q.reshape(-1, 1, 1, 64),
                            dst_q.reshape(-1, 1, 1, 64)],
                           axis=1).reshape(-1, 4, 2, 1, 64)

    deg_parts = _deg_kernel(dst_p)
    g1, deg = _tc1(x_p, W1, deg_parts)
    acc1 = _conv1(g1, g1, idx1)
    g2, g2h = _tc2(acc1, deg, b1, W2)
    acc2 = _conv2(g2, g2h, idx2)
    return _tcf(acc2, deg, b2, starts, Wd1, bd1, Wd2, bd2)


# speedup vs baseline: 1.2926x; 1.2926x over previous
import functools

import jax
import jax.numpy as jnp
from jax import lax
from jax.experimental import pallas as pl
from jax.experimental.pallas import tpu as pltpu
from jax.experimental.pallas import tpu_sc as plsc

N = 10000
NP = 10240
E = 320000
EP = 323584
B = 64
D_IN = 128
LAT = 128
HID = 256
MLP_HID = 512
OUT_DIM = 1275

NC, NS = 2, 16
NT = NP // NS
NH = NP // 2

_R = 256
_GI = NP // _R


def _sc_mesh():
    return plsc.VectorSubcoreMesh(
        core_axis_name="c", subcore_axis_name="s", num_cores=NC, num_subcores=NS
    )


_DEG_ET = EP // NS


@functools.partial(
    pl.kernel,
    out_type=jax.ShapeDtypeStruct((NC * NS, NH), jnp.float32),
    mesh=_sc_mesh(),
    compiler_params=pltpu.CompilerParams(needs_layout_passes=False),
    scratch_types=[
        pltpu.VMEM((_DEG_ET,), jnp.int32),
        pltpu.VMEM((NS * NH,), jnp.float32),
        pltpu.VMEM((NH,), jnp.float32),
    ],
)
def _deg_kernel(dst_hbm, out_hbm, dstbuf, cnt, red):
    c = lax.axis_index("c")
    t = lax.axis_index("s")
    lo = c * NH
    zeros16 = jnp.zeros((16,), jnp.float32)
    ones16 = jnp.ones((16,), jnp.float32)
    lane = lax.iota(jnp.int32, 16)

    def zbody(i, carry):
        cnt[pl.ds(i * 16, 16)] = zeros16
        return carry

    lax.fori_loop(0, NS * NH // 16, zbody, 0)

    pltpu.sync_copy(dst_hbm.at[pl.ds(t * _DEG_ET, _DEG_ET)], dstbuf)

    lane_off = lane * NH

    def body(j, carry):
        idx = dstbuf[pl.ds(j * 16, 16)]
        m = (idx >= lo) & (idx < lo + NH)
        loc = jnp.where(m, idx - lo, 0) + lane_off
        plsc.addupdate_scatter(cnt, [loc], ones16, mask=m)
        return carry

    lax.fori_loop(0, _DEG_ET // 16, body, 0)

    def rbody(i, carry):
        acc = cnt[pl.ds(i * 16, 16)]
        for r in range(1, NS):
            acc = acc + cnt[pl.ds(r * NH + i * 16, 16)]
        red[pl.ds(i * 16, 16)] = acc
        return carry

    lax.fori_loop(0, NH // 16, rbody, 0)

    pltpu.sync_copy(red, out_hbm.at[c * NS + t])


_K = 64
_NBI = 8
_NBR = 5


def _make_conv(epc, f, init_split):
    et = epc // NS
    ch = et // _K

    @functools.partial(
        pl.kernel,
        out_type=jax.ShapeDtypeStruct((NC * NP, f), jnp.float32),
        mesh=_sc_mesh(),
        compiler_params=pltpu.CompilerParams(needs_layout_passes=False),
        scratch_types=[
            pltpu.VMEM((_NBI, 2, 1, _K), jnp.int32),
            pltpu.VMEM((_NBR, _K, f), jnp.float32),
            pltpu.VMEM_SHARED((NP, f), jnp.float32),
            pltpu.SemaphoreType.DMA((_NBI,)),
            pltpu.SemaphoreType.DMA((_NBR,)),
            pltpu.SemaphoreType.DMA((_NBR,)),
        ],
    )
    def conv(g_hbm, ginit_hbm, idx_hbm, out_hbm,
             iring, rows, acc, isem, gsem, ssem):
        c = lax.axis_index("c")
        t = lax.axis_index("s")
        ioff = (c * NP if init_split else 0) + t * NT
        pltpu.sync_copy(ginit_hbm.at[pl.ds(ioff, NT)], acc.at[pl.ds(t * NT, NT)])
        cbase = (c * epc + t * et) // _K

        def issue_i(k):
            b = lax.rem(k, _NBI)
            pltpu.async_copy(idx_hbm.at[cbase + k], iring.at[b], isem.at[b])

        def wait_i(k):
            b = lax.rem(k, _NBI)
            pltpu.make_async_copy(idx_hbm.at[cbase + k], iring.at[b],
                                  isem.at[b]).wait()

        def issue_g(k):
            bi = lax.rem(k, _NBI)
            br = lax.rem(k, _NBR)
            pltpu.async_copy(g_hbm.at[iring.at[bi, 0, 0]], rows.at[br],
                             gsem.at[br])

        def wait_g(k):
            bi = lax.rem(k, _NBI)
            br = lax.rem(k, _NBR)
            pltpu.make_async_copy(g_hbm.at[iring.at[bi, 0, 0]], rows.at[br],
                                  gsem.at[br]).wait()

        def issue_s(k):
            bi = lax.rem(k, _NBI)
            br = lax.rem(k, _NBR)
            pltpu.async_copy(rows.at[br], acc.at[iring.at[bi, 1, 0]],
                             ssem.at[br], add=True)

        def wait_s(k):
            bi = lax.rem(k, _NBI)
            br = lax.rem(k, _NBR)
            pltpu.make_async_copy(rows.at[br], acc.at[iring.at[bi, 1, 0]],
                                  ssem.at[br]).wait()

        for k in range(_NBI - 2):
            issue_i(k)
        for k in range(3):
            wait_i(k)
            issue_g(k)
        plsc.subcore_barrier()

        def body(m, carry):
            wait_g(m)
            issue_s(m)

            @pl.when(m >= 2)
            def _():
                wait_s(m - 2)

            @pl.when(m + 3 < ch)
            def _():
                wait_i(m + 3)
                issue_g(m + 3)

            @pl.when(m + _NBI - 2 < ch)
            def _():
                issue_i(m + _NBI - 2)

            return carry

        lax.fori_loop(0, ch, body, 0)
        wait_s(ch - 2)
        wait_s(ch - 1)
        plsc.subcore_barrier()
        pltpu.sync_copy(acc.at[pl.ds(t * NT, NT)],
                        out_hbm.at[pl.ds(c * NP + t * NT, NT)])

    return conv


_conv1 = _make_conv(EP, D_IN, True)
_conv2 = _make_conv(EP // 2, LAT, True)


def _tc1_body(xb, w1b, dpb, g1b, degb):
    c = pl.program_id(1)
    deg = jnp.sum(dpb[...], axis=0) + 1.0
    dinv = lax.rsqrt(deg)
    h = jnp.dot(xb[...], w1b[...], preferred_element_type=jnp.float32)
    g1b[...] = h * dinv[:, None]

    @pl.when(c == 0)
    def _():
        degb[...] = deg


def _tc1(x_p, w1, deg_parts):
    return pl.pallas_call(
        _tc1_body,
        grid=(_GI, NC),
        in_specs=[
            pl.BlockSpec((_R, D_IN), lambda i, c: (i, 0)),
            pl.BlockSpec((D_IN, LAT), lambda i, c: (0, c)),
            pl.BlockSpec((NS, _R), lambda i, c: (i // 20, i % 20)),
        ],
        out_specs=[
            pl.BlockSpec((_R, LAT), lambda i, c: (c * _GI + i, 0)),
            pl.BlockSpec((_R,), lambda i, c: (i,)),
        ],
        out_shape=[
            jax.ShapeDtypeStruct((NC * NP, LAT), jnp.float32),
            jax.ShapeDtypeStruct((NP,), jnp.float32),
        ],
    )(x_p, w1, deg_parts)


def _tc2_body(aab, abb, degb, b1b, w2b, g2b, g2hb):
    dinv = lax.rsqrt(degb[...])
    a = jnp.concatenate([aab[...], abb[...]], axis=1)
    out1 = jnp.maximum(a * dinv[:, None] + b1b[...][None, :], 0.0)
    h2 = jnp.dot(out1, w2b[...], preferred_element_type=jnp.float32)
    g2 = h2 * dinv[:, None]
    g2b[...] = g2
    g2hb[...] = 0.5 * g2


def _tc2(acc1, deg, b1, w2):
    return pl.pallas_call(
        _tc2_body,
        grid=(_GI,),
        in_specs=[
            pl.BlockSpec((_R, D_IN), lambda i: (i, 0)),
            pl.BlockSpec((_R, D_IN), lambda i: (_GI + i, 0)),
            pl.BlockSpec((_R,), lambda i: (i,)),
            pl.BlockSpec((HID,), lambda i: (0,)),
            pl.BlockSpec((HID, LAT), lambda i: (0, 0)),
        ],
        out_specs=[
            pl.BlockSpec((_R, LAT), lambda i: (i, 0)),
            pl.BlockSpec((_R, LAT), lambda i: (i, 0)),
        ],
        out_shape=[
            jax.ShapeDtypeStruct((NP, LAT), jnp.float32),
            jax.ShapeDtypeStruct((NP, LAT), jnp.float32),
        ],
    )(acc1, acc1, deg, b1, w2)


def _tcf_body(acc2r, degr, b2r, startsr, wd1r, bd1r, wd2r, bd2r, outr, hfin):
    dinv = lax.rsqrt(degr[...])
    a = acc2r[pl.ds(0, NP), :] + acc2r[pl.ds(NP, NP), :]
    hfin[...] = a * dinv[:, None] + b2r[...][None, :]

    neg = jnp.float32(-3.0e38)

    def seg(b, z):
        s0 = startsr[b]
        s1 = startsr[b + 1]
        j0 = s0 // 8
        nb = (s1 - j0 * 8 + 7) // 8

        def blk(j, acc8):
            off = pl.multiple_of((j0 + j) * 8, 8)
            rows = hfin[pl.ds(off, 8), :]
            rid = off + lax.broadcasted_iota(jnp.int32, (8, 1), 0)
            keep = (rid >= s0) & (rid < s1)
            return jnp.maximum(acc8, jnp.where(keep, rows, neg))

        acc8 = lax.fori_loop(0, nb, blk, jnp.full((8, LAT), neg, jnp.float32))
        zrow = jnp.max(acc8, axis=0)
        sel = lax.broadcasted_iota(jnp.int32, (B, 1), 0) == b
        return jnp.where(sel, zrow[None, :], z)

    z = lax.fori_loop(0, B, seg, jnp.full((B, LAT), neg, jnp.float32))
    y = jnp.maximum(
        jnp.dot(z, wd1r[...], preferred_element_type=jnp.float32)
        + bd1r[...][None, :], 0.0)
    o = (jnp.dot(y, wd2r[...], preferred_element_type=jnp.float32)
         + bd2r[...][None, :])
    outr[...] = jax.nn.sigmoid(o)


def _tcf(acc2, deg, b2, starts, wd1, bd1, wd2, bd2):
    return pl.pallas_call(
        _tcf_body,
        in_specs=[
            pl.BlockSpec(memory_space=pltpu.VMEM),
            pl.BlockSpec(memory_space=pltpu.VMEM),
            pl.BlockSpec(memory_space=pltpu.VMEM),
            pl.BlockSpec(memory_space=pltpu.SMEM),
            pl.BlockSpec(memory_space=pltpu.VMEM),
            pl.BlockSpec(memory_space=pltpu.VMEM),
            pl.BlockSpec(memory_space=pltpu.VMEM),
            pl.BlockSpec(memory_space=pltpu.VMEM),
        ],
        out_shape=jax.ShapeDtypeStruct((B, OUT_DIM), jnp.float32),
        scratch_shapes=[pltpu.VMEM((NP, LAT), jnp.float32)],
    )(acc2, deg, b2, starts, wd1, bd1, wd2, bd2)


def kernel(x, edge_index, batch, W1, b1, W2, b2, Wd1, bd1, Wd2, bd2):
    src = edge_index[0]
    dst = edge_index[1]
    pad = EP - E
    src_p = jnp.concatenate([src, jnp.zeros((pad,), jnp.int32)])
    trash = N + jnp.arange(pad, dtype=jnp.int32) % (NP - N)
    dst_p = jnp.concatenate([dst, trash])
    src2 = jnp.concatenate([src_p, src_p + NP])
    dst2 = jnp.concatenate([dst_p, dst_p])
    x_p = jnp.pad(x, ((0, NP - N), (0, 0)))
    starts = jnp.searchsorted(batch, jnp.arange(B + 1, dtype=jnp.int32)
                              ).astype(jnp.int32)

    idx1 = jnp.concatenate([src2.reshape(-1, 1, 1, 64),
                            dst2.reshape(-1, 1, 1, 64)], axis=1)
    idx2 = jnp.concatenate([src_p.reshape(-1, 1, 1, 64),
                            dst_p.reshape(-1, 1, 1, 64)], axis=1)

    deg_parts = _deg_kernel(dst_p)
    g1, deg = _tc1(x_p, W1, deg_parts)
    acc1 = _conv1(g1, g1, idx1)
    g2, g2h = _tc2(acc1, deg, b1, W2)
    ginit2 = jnp.concatenate([g2, jnp.zeros_like(g2)])
    acc2 = _conv2(g2, ginit2, idx2)
    return _tcf(acc2, deg, b2, starts, Wd1, bd1, Wd2, bd2)


# speedup vs baseline: 1.3182x; 1.0199x over previous
import functools

import jax
import jax.numpy as jnp
from jax import lax
from jax.experimental import pallas as pl
from jax.experimental.pallas import tpu as pltpu
from jax.experimental.pallas import tpu_sc as plsc

N = 10000
NP = 10240
E = 320000
EP = 323584
B = 64
D_IN = 128
LAT = 128
HID = 256
MLP_HID = 512
OUT_DIM = 1275

NC, NS = 2, 16
NT = NP // NS
NH = NP // 2

_R = 256
_GI = NP // _R


def _sc_mesh():
    return plsc.VectorSubcoreMesh(
        core_axis_name="c", subcore_axis_name="s", num_cores=NC, num_subcores=NS
    )


_DEG_ET = EP // NS


@functools.partial(
    pl.kernel,
    out_type=jax.ShapeDtypeStruct((NC * NS, NH), jnp.float32),
    mesh=_sc_mesh(),
    compiler_params=pltpu.CompilerParams(needs_layout_passes=False),
    scratch_types=[
        pltpu.VMEM((_DEG_ET,), jnp.int32),
        pltpu.VMEM((NS * NH,), jnp.float32),
        pltpu.VMEM((NH,), jnp.float32),
    ],
)
def _deg_kernel(dst_hbm, out_hbm, dstbuf, cnt, red):
    c = lax.axis_index("c")
    t = lax.axis_index("s")
    lo = c * NH
    zeros16 = jnp.zeros((16,), jnp.float32)
    ones16 = jnp.ones((16,), jnp.float32)
    lane = lax.iota(jnp.int32, 16)

    def zbody(i, carry):
        cnt[pl.ds(i * 16, 16)] = zeros16
        return carry

    lax.fori_loop(0, NS * NH // 16, zbody, 0)

    pltpu.sync_copy(dst_hbm.at[pl.ds(t * _DEG_ET, _DEG_ET)], dstbuf)

    lane_off = lane * NH

    def body(j, carry):
        idx = dstbuf[pl.ds(j * 16, 16)]
        m = (idx >= lo) & (idx < lo + NH)
        loc = jnp.where(m, idx - lo, 0) + lane_off
        plsc.addupdate_scatter(cnt, [loc], ones16, mask=m)
        return carry

    lax.fori_loop(0, _DEG_ET // 16, body, 0)

    def rbody(i, carry):
        acc = cnt[pl.ds(i * 16, 16)]
        for r in range(1, NS):
            acc = acc + cnt[pl.ds(r * NH + i * 16, 16)]
        red[pl.ds(i * 16, 16)] = acc
        return carry

    lax.fori_loop(0, NH // 16, rbody, 0)

    pltpu.sync_copy(red, out_hbm.at[c * NS + t])


_K = 64
_NBI = 8
_NBR = 5


def _make_conv(epc, f, init_split):
    et = epc // NS
    ch = et // _K

    @functools.partial(
        pl.kernel,
        out_type=jax.ShapeDtypeStruct((NC * NP, f), jnp.float32),
        mesh=_sc_mesh(),
        compiler_params=pltpu.CompilerParams(needs_layout_passes=False),
        scratch_types=[
            pltpu.VMEM((_NBI, 2, 1, _K), jnp.int32),
            pltpu.VMEM((_NBR, _K, f), jnp.float32),
            pltpu.VMEM_SHARED((NP, f), jnp.float32),
            pltpu.SemaphoreType.DMA((_NBI,)),
            pltpu.SemaphoreType.DMA((_NBR,)),
        ],
    )
    def conv(g_hbm, ginit_hbm, idx_hbm, out_hbm,
             iring, rows, acc, isem, gsem):
        c = lax.axis_index("c")
        t = lax.axis_index("s")
        ioff = (c * NP if init_split else 0) + t * NT
        pltpu.sync_copy(ginit_hbm.at[pl.ds(ioff, NT)], acc.at[pl.ds(t * NT, NT)])
        cbase = (c * epc + t * et) // _K

        def issue_i(k):
            b = lax.rem(k, _NBI)
            pltpu.async_copy(idx_hbm.at[cbase + k], iring.at[b], isem.at[b])

        def wait_i(k):
            b = lax.rem(k, _NBI)
            pltpu.make_async_copy(idx_hbm.at[cbase + k], iring.at[b],
                                  isem.at[b]).wait()

        def issue_g(k):
            bi = lax.rem(k, _NBI)
            br = lax.rem(k, _NBR)
            pltpu.async_copy(g_hbm.at[iring.at[bi, 0, 0]], rows.at[br],
                             gsem.at[br])

        def wait_g(k):
            bi = lax.rem(k, _NBI)
            br = lax.rem(k, _NBR)
            pltpu.make_async_copy(g_hbm.at[iring.at[bi, 0, 0]], rows.at[br],
                                  gsem.at[br]).wait()

        def scatter(k):
            bi = lax.rem(k, _NBI)
            br = lax.rem(k, _NBR)
            pltpu.sync_copy(rows.at[br], acc.at[iring.at[bi, 1, 0]], add=True)

        for k in range(_NBI - 1):
            issue_i(k)
        for k in range(4):
            wait_i(k)
            issue_g(k)
        plsc.subcore_barrier()

        def body(m, carry):
            wait_g(m)
            scatter(m)

            @pl.when(m + 4 < ch)
            def _():
                wait_i(m + 4)
                issue_g(m + 4)

            @pl.when(m + _NBI - 1 < ch)
            def _():
                issue_i(m + _NBI - 1)

            return carry

        lax.fori_loop(0, ch, body, 0)
        plsc.subcore_barrier()
        pltpu.sync_copy(acc.at[pl.ds(t * NT, NT)],
                        out_hbm.at[pl.ds(c * NP + t * NT, NT)])

    return conv


_conv1 = _make_conv(EP, D_IN, True)
_conv2 = _make_conv(EP // 2, LAT, True)


def _tc1_body(xb, w1b, dpb, g1b, degb):
    c = pl.program_id(1)
    deg = jnp.sum(dpb[...], axis=0) + 1.0
    dinv = lax.rsqrt(deg)
    h = jnp.dot(xb[...], w1b[...], preferred_element_type=jnp.float32)
    g1b[...] = h * dinv[:, None]

    @pl.when(c == 0)
    def _():
        degb[...] = deg


def _tc1(x_p, w1, deg_parts):
    return pl.pallas_call(
        _tc1_body,
        grid=(_GI, NC),
        in_specs=[
            pl.BlockSpec((_R, D_IN), lambda i, c: (i, 0)),
            pl.BlockSpec((D_IN, LAT), lambda i, c: (0, c)),
            pl.BlockSpec((NS, _R), lambda i, c: (i // 20, i % 20)),
        ],
        out_specs=[
            pl.BlockSpec((_R, LAT), lambda i, c: (c * _GI + i, 0)),
            pl.BlockSpec((_R,), lambda i, c: (i,)),
        ],
        out_shape=[
            jax.ShapeDtypeStruct((NC * NP, LAT), jnp.float32),
            jax.ShapeDtypeStruct((NP,), jnp.float32),
        ],
    )(x_p, w1, deg_parts)


def _tc2_body(aab, abb, degb, b1b, w2b, g2b, g2hb):
    dinv = lax.rsqrt(degb[...])
    a = jnp.concatenate([aab[...], abb[...]], axis=1)
    out1 = jnp.maximum(a * dinv[:, None] + b1b[...][None, :], 0.0)
    h2 = jnp.dot(out1, w2b[...], preferred_element_type=jnp.float32)
    g2 = h2 * dinv[:, None]
    g2b[...] = g2
    g2hb[...] = 0.5 * g2


def _tc2(acc1, deg, b1, w2):
    return pl.pallas_call(
        _tc2_body,
        grid=(_GI,),
        in_specs=[
            pl.BlockSpec((_R, D_IN), lambda i: (i, 0)),
            pl.BlockSpec((_R, D_IN), lambda i: (_GI + i, 0)),
            pl.BlockSpec((_R,), lambda i: (i,)),
            pl.BlockSpec((HID,), lambda i: (0,)),
            pl.BlockSpec((HID, LAT), lambda i: (0, 0)),
        ],
        out_specs=[
            pl.BlockSpec((_R, LAT), lambda i: (i, 0)),
            pl.BlockSpec((_R, LAT), lambda i: (i, 0)),
        ],
        out_shape=[
            jax.ShapeDtypeStruct((NP, LAT), jnp.float32),
            jax.ShapeDtypeStruct((NP, LAT), jnp.float32),
        ],
    )(acc1, acc1, deg, b1, w2)


def _tcf_body(acc2r, degr, b2r, startsr, wd1r, bd1r, wd2r, bd2r, outr, hfin):
    dinv = lax.rsqrt(degr[...])
    a = acc2r[pl.ds(0, NP), :] + acc2r[pl.ds(NP, NP), :]
    hfin[...] = a * dinv[:, None] + b2r[...][None, :]

    neg = jnp.float32(-3.0e38)

    def seg(b, z):
        s0 = startsr[b]
        s1 = startsr[b + 1]
        j0 = s0 // 8
        nb = (s1 - j0 * 8 + 7) // 8

        def blk(j, acc8):
            off = pl.multiple_of((j0 + j) * 8, 8)
            rows = hfin[pl.ds(off, 8), :]
            rid = off + lax.broadcasted_iota(jnp.int32, (8, 1), 0)
            keep = (rid >= s0) & (rid < s1)
            return jnp.maximum(acc8, jnp.where(keep, rows, neg))

        acc8 = lax.fori_loop(0, nb, blk, jnp.full((8, LAT), neg, jnp.float32))
        zrow = jnp.max(acc8, axis=0)
        sel = lax.broadcasted_iota(jnp.int32, (B, 1), 0) == b
        return jnp.where(sel, zrow[None, :], z)

    z = lax.fori_loop(0, B, seg, jnp.full((B, LAT), neg, jnp.float32))
    y = jnp.maximum(
        jnp.dot(z, wd1r[...], preferred_element_type=jnp.float32)
        + bd1r[...][None, :], 0.0)
    o = (jnp.dot(y, wd2r[...], preferred_element_type=jnp.float32)
         + bd2r[...][None, :])
    outr[...] = jax.nn.sigmoid(o)


def _tcf(acc2, deg, b2, starts, wd1, bd1, wd2, bd2):
    return pl.pallas_call(
        _tcf_body,
        in_specs=[
            pl.BlockSpec(memory_space=pltpu.VMEM),
            pl.BlockSpec(memory_space=pltpu.VMEM),
            pl.BlockSpec(memory_space=pltpu.VMEM),
            pl.BlockSpec(memory_space=pltpu.SMEM),
            pl.BlockSpec(memory_space=pltpu.VMEM),
            pl.BlockSpec(memory_space=pltpu.VMEM),
            pl.BlockSpec(memory_space=pltpu.VMEM),
            pl.BlockSpec(memory_space=pltpu.VMEM),
        ],
        out_shape=jax.ShapeDtypeStruct((B, OUT_DIM), jnp.float32),
        scratch_shapes=[pltpu.VMEM((NP, LAT), jnp.float32)],
    )(acc2, deg, b2, starts, wd1, bd1, wd2, bd2)


def kernel(x, edge_index, batch, W1, b1, W2, b2, Wd1, bd1, Wd2, bd2):
    src = edge_index[0]
    dst = edge_index[1]
    pad = EP - E
    src_p = jnp.concatenate([src, jnp.zeros((pad,), jnp.int32)])
    trash = N + jnp.arange(pad, dtype=jnp.int32) % (NP - N)
    dst_p = jnp.concatenate([dst, trash])
    src2 = jnp.concatenate([src_p, src_p + NP])
    dst2 = jnp.concatenate([dst_p, dst_p])
    x_p = jnp.pad(x, ((0, NP - N), (0, 0)))
    starts = jnp.searchsorted(batch, jnp.arange(B + 1, dtype=jnp.int32)
                              ).astype(jnp.int32)

    idx1 = jnp.concatenate([src2.reshape(-1, 1, 1, 64),
                            dst2.reshape(-1, 1, 1, 64)], axis=1)
    idx2 = jnp.concatenate([src_p.reshape(-1, 1, 1, 64),
                            dst_p.reshape(-1, 1, 1, 64)], axis=1)

    deg_parts = _deg_kernel(dst_p)
    g1, deg = _tc1(x_p, W1, deg_parts)
    acc1 = _conv1(g1, g1, idx1)
    g2, g2h = _tc2(acc1, deg, b1, W2)
    ginit2 = jnp.concatenate([g2, jnp.zeros_like(g2)])
    acc2 = _conv2(g2, ginit2, idx2)
    return _tcf(acc2, deg, b2, starts, Wd1, bd1, Wd2, bd2)
